# baseline (speedup 1.0000x reference)
import functools

import jax
import jax.numpy as jnp
from jax import lax
from jax.experimental import pallas as pl
from jax.experimental.pallas import tpu as pltpu
from jax.experimental.pallas import tpu_sc as plsc

_LANES = 16
_CHUNK = 128


def _gmf_call(B, D, n_workers, v3, h3, virus_w, human_w, vb_w, hb_w, bias16):
    per_w = B // n_workers
    n_chunks = per_w // _CHUNK
    n_groups = per_w // _LANES
    mesh = plsc.VectorSubcoreMesh(core_axis_name="c", subcore_axis_name="s")

    @functools.partial(
        pl.kernel,
        mesh=mesh,
        out_type=jax.ShapeDtypeStruct((B,), jnp.float32),
        scratch_types=[
            pltpu.VMEM((n_chunks, _CHUNK), jnp.int32),
            pltpu.VMEM((n_chunks, _CHUNK), jnp.int32),
            pltpu.VMEM((per_w, D), jnp.float32),
            pltpu.VMEM((per_w, D), jnp.float32),
            pltpu.VMEM((per_w,), jnp.float32),
            pltpu.VMEM((per_w,), jnp.float32),
            pltpu.VMEM((_LANES,), jnp.float32),
            pltpu.VMEM((per_w,), jnp.float32),
            pltpu.SemaphoreType.DMA,
        ],
        compiler_params=pltpu.CompilerParams(
            needs_layout_passes=False, use_tc_tiling_on_sc=False),
    )
    def body(v_hbm, h_hbm, vw_hbm, hw_hbm, vb_hbm, hb_hbm, bias_hbm, out_hbm,
             vidx, hidx, u_rows, v_rows, bu, bv, bias_v, out_v, sem):
        num_c = lax.axis_size("c")
        wid = lax.axis_index("s") * num_c + lax.axis_index("c")

        pltpu.sync_copy(v_hbm.at[wid], vidx)
        pltpu.sync_copy(h_hbm.at[wid], hidx)
        pltpu.sync_copy(bias_hbm, bias_v)

        u2d = u_rows
        v2d = v_rows
        copies = []
        for j in range(n_chunks):
            rows = pl.ds(j * _CHUNK, _CHUNK)
            copies.append(pltpu.async_copy(vw_hbm.at[vidx.at[j]],
                                           u2d.at[rows], sem))
            copies.append(pltpu.async_copy(hw_hbm.at[hidx.at[j]],
                                           v2d.at[rows], sem))
            copies.append(pltpu.async_copy(vb_hbm.at[vidx.at[j]],
                                           bu.at[rows], sem))
            copies.append(pltpu.async_copy(hb_hbm.at[hidx.at[j]],
                                           bv.at[rows], sem))
        for cp in copies:
            cp.wait()

        iota = lax.iota(jnp.int32, _LANES)
        bias_vec = bias_v[...]
        u_flat = u_rows
        v_flat = v_rows

        def group(g, carry):
            row = g * _LANES + iota
            lanes = pl.ds(g * _LANES, _LANES)
            acc = bias_vec + bu[lanes] + bv[lanes]
            for d in range(D):
                col = jnp.full((_LANES,), d, jnp.int32)
                ug = plsc.load_gather(u2d, [row, col])
                vg = plsc.load_gather(v2d, [row, col])
                acc = acc + ug * vg
            out_v[pl.ds(g * _LANES, _LANES)] = acc
            return carry

        lax.fori_loop(0, n_groups, group, 0)
        pltpu.sync_copy(out_v, out_hbm.at[pl.ds(wid * per_w, per_w)])

    return body(v3, h3, virus_w, human_w, vb_w, hb_w, bias16)


def kernel(v_idxs, h_idxs, virus_w, human_w, vb_w, hb_w, bias):
    B = v_idxs.shape[0]
    D = virus_w.shape[1]
    info = plsc.get_sparse_core_info()
    n_workers = info.num_cores * info.num_subcores
    n_chunks = B // n_workers // _CHUNK
    v3 = v_idxs.astype(jnp.int32).reshape(n_workers, n_chunks, _CHUNK)
    h3 = h_idxs.astype(jnp.int32).reshape(n_workers, n_chunks, _CHUNK)
    bias16 = jnp.broadcast_to(bias.astype(jnp.float32), (_LANES,))
    return _gmf_call(B, D, n_workers, v3, h3, virus_w, human_w,
                     vb_w.reshape(-1), hb_w.reshape(-1), bias16)
